# baseline (speedup 1.0000x reference)
import functools

import jax
import jax.numpy as jnp
import numpy as np
from jax import lax
from jax.experimental import pallas as pl
from jax.experimental.pallas import tpu as pltpu
from jax.experimental.pallas import tpu_sc as plsc

B, F, D = 16384, 26, 16
V = 2600000
N = B * F
_info = plsc.get_sparse_core_info()
NC, NS, L = _info.num_cores, _info.num_subcores, _info.num_lanes
NW = NC * NS

SLAB = 1024
NFULL = (V // 128) // 8
VREM0 = NFULL * SLAB
VREM = V - VREM0
APW = -(-NFULL // NW)

NPW = N // NW
PERIOD = (F * L) // np.gcd(F, L)
NSEG = PERIOD // L
NITER = NPW // PERIOD
CHUNK = 3328
NCHUNK = NPW // CHUNK

_FIELD_DIMS = [100000] * F
_OFFSETS = np.concatenate([[0], np.cumsum(_FIELD_DIMS)[:-1]]).astype(np.int32)
_PATTERN = _OFFSETS[np.arange(PERIOD) % F]


def _transpose_kernel(tt_hbm, trem_hbm, tlin_hbm, in0, in1, ob0, ob1,
                      rsem0, rsem1, wsem0, wsem1):
    wid = lax.axis_index("s") * NC + lax.axis_index("c")
    c0 = wid * APW
    end = jnp.minimum(c0 + APW, NFULL)
    iota = lax.iota(jnp.int32, L)

    ins = (in0, in1)
    obs = (ob0, ob1)
    rsems = (rsem0, rsem1)
    wsems = (wsem0, wsem1)

    def read(c, p):
        pltpu.async_copy(tt_hbm.at[:, pl.ds(c * SLAB, SLAB)], ins[p], rsems[p])

    def wait_read(p):
        pltpu.make_async_copy(tt_hbm.at[:, pl.ds(0, SLAB)], ins[p],
                              rsems[p]).wait()

    def write(c, p):
        pltpu.async_copy(obs[p], tlin_hbm.at[pl.ds(c * SLAB * D, SLAB * D)],
                         wsems[p])

    def wait_write(p):
        pltpu.make_async_copy(obs[p], tlin_hbm.at[pl.ds(0, SLAB * D)],
                              wsems[p]).wait()

    def transpose(p):
        inb, ob = ins[p], obs[p]
        @plsc.parallel_loop(0, SLAB // L, unroll=8)
        def _(j):
            jbase = iota * D + j * (L * D)
            for d in range(D):
                row = inb[d, pl.ds(j * L, L)]
                plsc.store_scatter(ob, [jbase + d], row)

    @pl.when(c0 < end)
    def _():
        read(c0, 0)

    def pair(k, carry):
        c = c0 + 2 * k

        @pl.when(c < end)
        def _():
            @pl.when(c + 1 < end)
            def _():
                read(c + 1, 1)
            wait_read(0)
            @pl.when(k > 0)
            def _():
                wait_write(0)
            transpose(0)
            write(c, 0)

        @pl.when(c + 1 < end)
        def _():
            @pl.when(c + 2 < end)
            def _():
                read(c + 2, 0)
            wait_read(1)
            @pl.when(k > 0)
            def _():
                wait_write(1)
            transpose(1)
            write(c + 1, 1)
        return carry

    lax.fori_loop(0, (APW + 1) // 2, pair, 0)

    nmine = jnp.maximum(end - c0, 0)
    @pl.when(nmine >= 1)
    def _():
        wait_write(0)
    @pl.when(nmine >= 2)
    def _():
        wait_write(1)

    @pl.when(wid == NW - 1)
    def _():
        pltpu.sync_copy(trem_hbm, ob0.at[pl.ds(0, VREM * D)])
        pltpu.sync_copy(ob0.at[pl.ds(0, VREM * D)],
                        tlin_hbm.at[pl.ds(VREM0 * D, VREM * D)])


@jax.jit
def _relayout(tt, trem):
    return pl.kernel(
        _transpose_kernel,
        out_type=jax.ShapeDtypeStruct((V * D,), jnp.float32),
        mesh=plsc.VectorSubcoreMesh(core_axis_name="c", subcore_axis_name="s"),
        scratch_types=[
            pltpu.VMEM((L, SLAB), jnp.float32),
            pltpu.VMEM((L, SLAB), jnp.float32),
            pltpu.VMEM((SLAB * D,), jnp.float32),
            pltpu.VMEM((SLAB * D,), jnp.float32),
            pltpu.SemaphoreType.DMA,
            pltpu.SemaphoreType.DMA,
            pltpu.SemaphoreType.DMA,
            pltpu.SemaphoreType.DMA,
        ],
        compiler_params=pltpu.CompilerParams(needs_layout_passes=False),
    )(tt, trem)


def _lookup_kernel(x_hbm, patt_hbm, t2d_hbm, out_hbm, idx_v, patt_v, rows_v,
                   sem):
    wid = lax.axis_index("s") * NC + lax.axis_index("c")
    base = wid * NPW

    pltpu.sync_copy(x_hbm.at[pl.ds(base, NPW)], idx_v)
    pltpu.sync_copy(patt_hbm, patt_v)

    pregs = [patt_v[pl.ds(u * L, L)] for u in range(NSEG)]

    def add_offsets(t, carry):
        s = t * PERIOD
        for u in range(NSEG):
            sl = pl.ds(s + u * L, L)
            idx_v[sl] = idx_v[sl] + pregs[u]
        return carry

    lax.fori_loop(0, NITER, add_offsets, 0)

    def do_chunk(k, carry):
        pltpu.async_copy(
            t2d_hbm.at[idx_v.at[pl.ds(k * CHUNK, CHUNK)]], rows_v, sem
        ).wait()
        pltpu.sync_copy(rows_v, out_hbm.at[pl.ds(base + k * CHUNK, CHUNK)])
        return carry

    lax.fori_loop(0, NCHUNK, do_chunk, 0)


@jax.jit
def _lookup(x_flat, patt, t2d):
    return pl.kernel(
        _lookup_kernel,
        out_type=jax.ShapeDtypeStruct((N, D), jnp.float32),
        mesh=plsc.VectorSubcoreMesh(core_axis_name="c", subcore_axis_name="s"),
        scratch_types=[
            pltpu.VMEM((NPW,), jnp.int32),
            pltpu.VMEM((PERIOD,), jnp.int32),
            pltpu.VMEM((CHUNK, D), jnp.float32),
            pltpu.SemaphoreType.DMA,
        ],
        compiler_params=pltpu.CompilerParams(use_tc_tiling_on_sc=False),
    )(x_flat, patt, t2d)


def kernel(x, table):
    patt = jnp.asarray(_PATTERN)
    trem = table[VREM0:].reshape(-1)
    tlin = _relayout(table.T, trem)
    t2d = tlin.reshape(V, D)
    out = _lookup(x.reshape(-1), patt, t2d)
    return out.reshape(B, F, D)


# speedup vs baseline: 2.2477x; 2.2477x over previous
import functools

import jax
import jax.numpy as jnp
import numpy as np
from jax import lax
from jax.experimental import pallas as pl
from jax.experimental.pallas import tpu as pltpu
from jax.experimental.pallas import tpu_sc as plsc

B, F, D = 16384, 26, 16
V = 2600000
_info = plsc.get_sparse_core_info()
NC, NS, L = _info.num_cores, _info.num_subcores, _info.num_lanes
NW = NC * NS

SLAB = 1024
NFULL = (V // 128) // 8
VREM0 = NFULL * SLAB
VREM = V - VREM0
APW = -(-NFULL // NW)

NFG = 2
FPG = F // NFG
NBS = NW // NFG
BPW = B // NBS
DG = D // 8
PLANE = (B // 128) * 8 * 128
WSLICE = (BPW // 128) * 8 * 128

_FIELD_DIMS = [100000] * F
_OFFSETS = np.concatenate([[0], np.cumsum(_FIELD_DIMS)[:-1]]).astype(np.int32)
_OFF_TAB = np.repeat(_OFFSETS[:, None], L, axis=1)
_BASEVEC = ((np.arange(L) // 8) * (8 * BPW)
            + (np.arange(L) % 8) * 128).astype(np.int32)
_OFF_ARG = np.concatenate([_OFF_TAB, _BASEVEC[None, :]], axis=0)


def _transpose_kernel(tt_hbm, trem_hbm, tlin_hbm, in0, in1, ob0, ob1,
                      rsem0, rsem1, wsem0, wsem1):
    wid = lax.axis_index("s") * NC + lax.axis_index("c")
    c0 = wid * APW
    end = jnp.minimum(c0 + APW, NFULL)
    iota = lax.iota(jnp.int32, L)

    ins = (in0, in1)
    obs = (ob0, ob1)
    rsems = (rsem0, rsem1)
    wsems = (wsem0, wsem1)

    def read(c, p):
        pltpu.async_copy(tt_hbm.at[:, pl.ds(c * SLAB, SLAB)], ins[p], rsems[p])

    def wait_read(p):
        pltpu.make_async_copy(tt_hbm.at[:, pl.ds(0, SLAB)], ins[p],
                              rsems[p]).wait()

    def write(c, p):
        pltpu.async_copy(obs[p], tlin_hbm.at[pl.ds(c * SLAB * D, SLAB * D)],
                         wsems[p])

    def wait_write(p):
        pltpu.make_async_copy(obs[p], tlin_hbm.at[pl.ds(0, SLAB * D)],
                              wsems[p]).wait()

    def transpose(p):
        inb, ob = ins[p], obs[p]
        @plsc.parallel_loop(0, SLAB // L, unroll=4)
        def _(j):
            jbase = iota * D + j * (L * D)
            for d in range(D):
                row = inb[d, pl.ds(j * L, L)]
                plsc.store_scatter(ob, [jbase + d], row)

    @pl.when(c0 < end)
    def _():
        read(c0, 0)

    def pair(k, carry):
        c = c0 + 2 * k

        @pl.when(c < end)
        def _():
            @pl.when(c + 1 < end)
            def _():
                read(c + 1, 1)
            wait_read(0)
            @pl.when(k > 0)
            def _():
                wait_write(0)
            transpose(0)
            write(c, 0)

        @pl.when(c + 1 < end)
        def _():
            @pl.when(c + 2 < end)
            def _():
                read(c + 2, 0)
            wait_read(1)
            @pl.when(k > 0)
            def _():
                wait_write(1)
            transpose(1)
            write(c + 1, 1)
        return carry

    lax.fori_loop(0, (APW + 1) // 2, pair, 0)

    nmine = jnp.maximum(end - c0, 0)
    @pl.when(nmine >= 1)
    def _():
        wait_write(0)
    @pl.when(nmine >= 2)
    def _():
        wait_write(1)

    @pl.when(wid == NW - 1)
    def _():
        pltpu.sync_copy(trem_hbm, ob0.at[pl.ds(0, VREM * D)])
        pltpu.sync_copy(ob0.at[pl.ds(0, VREM * D)],
                        tlin_hbm.at[pl.ds(VREM0 * D, VREM * D)])


@jax.jit
def _relayout(tt, trem):
    return pl.kernel(
        _transpose_kernel,
        out_type=jax.ShapeDtypeStruct((V * D,), jnp.float32),
        mesh=plsc.VectorSubcoreMesh(core_axis_name="c", subcore_axis_name="s"),
        scratch_types=[
            pltpu.VMEM((L, SLAB), jnp.float32),
            pltpu.VMEM((L, SLAB), jnp.float32),
            pltpu.VMEM((SLAB * D,), jnp.float32),
            pltpu.VMEM((SLAB * D,), jnp.float32),
            pltpu.SemaphoreType.DMA,
            pltpu.SemaphoreType.DMA,
            pltpu.SemaphoreType.DMA,
            pltpu.SemaphoreType.DMA,
        ],
        compiler_params=pltpu.CompilerParams(needs_layout_passes=False),
    )(tt, trem)


def _lookup_kernel(xt_hbm, off_hbm, t2d_hbm, out_hbm, idx_v, off_v, bv_v,
                   r0, r1, t0, t1, gsem0, gsem1, wsem0, wsem1):
    wid = lax.axis_index("s") * NC + lax.axis_index("c")
    fg = wid // NBS
    bs = wid % NBS
    f0 = fg * FPG
    b0 = bs * BPW
    pltpu.sync_copy(xt_hbm.at[pl.ds(f0, FPG), pl.ds(b0, BPW)], idx_v)
    pltpu.sync_copy(off_hbm.at[pl.ds(f0, FPG)], off_v)
    pltpu.sync_copy(off_hbm.at[pl.ds(F, 1)], bv_v)
    basevec = bv_v[0, :]

    def add_off(fi, carry):
        off_reg = off_v[fi, :]
        def body(j, c):
            sl = pl.ds(j * L, L)
            idx_v[fi, sl] = idx_v[fi, sl] + off_reg
            return c
        return lax.fori_loop(0, BPW // L, body, carry)

    lax.fori_loop(0, FPG, add_off, 0)

    rbufs = (r0, r1)
    tbufs = (t0, t1)
    gsems = (gsem0, gsem1)
    wsems = (wsem0, wsem1)

    def fire_gather(fi, p):
        pltpu.async_copy(t2d_hbm.at[idx_v.at[fi]], rbufs[p], gsems[p])

    def wait_gather(p):
        pltpu.make_async_copy(t2d_hbm.at[idx_v.at[0]], rbufs[p],
                              gsems[p]).wait()

    def fire_writes(fi, p):
        for dg in range(DG):
            pltpu.async_copy(
                tbufs[p].at[pl.ds(dg * WSLICE, WSLICE)],
                out_hbm.at[f0 + fi, dg, pl.ds(bs * WSLICE, WSLICE)],
                wsems[p])

    def wait_writes(p):
        for dg in range(DG):
            pltpu.make_async_copy(
                tbufs[p].at[pl.ds(dg * WSLICE, WSLICE)],
                out_hbm.at[0, dg, pl.ds(0, WSLICE)], wsems[p]).wait()

    def transpose(p):
        rb, tb = rbufs[p], tbufs[p]
        def bt_body(bt, carry):
            def bl_body(q, c):
                for u in range(4):
                    bl = q * 4 + u
                    row = rb[bt * 128 + bl, :]
                    plsc.store_scatter(tb, [basevec + (bt * 1024 + bl)], row)
                return c
            return lax.fori_loop(0, 32, bl_body, carry)
        lax.fori_loop(0, BPW // 128, bt_body, 0)

    fire_gather(0, 0)
    for fi in range(FPG):
        p = fi % 2
        if fi + 1 < FPG:
            fire_gather(fi + 1, (fi + 1) % 2)
        wait_gather(p)
        if fi >= 2:
            wait_writes(p)
        transpose(p)
        fire_writes(fi, p)
    wait_writes((FPG - 1) % 2)
    wait_writes((FPG - 2) % 2)


@jax.jit
def _lookup(xt, off, t2d):
    return pl.kernel(
        _lookup_kernel,
        out_type=jax.ShapeDtypeStruct((F, DG, PLANE), jnp.float32),
        mesh=plsc.VectorSubcoreMesh(core_axis_name="c", subcore_axis_name="s"),
        scratch_types=[
            pltpu.VMEM((FPG, BPW), jnp.int32),
            pltpu.VMEM((FPG, L), jnp.int32),
            pltpu.VMEM((1, L), jnp.int32),
            pltpu.VMEM((BPW, D), jnp.float32),
            pltpu.VMEM((BPW, D), jnp.float32),
            pltpu.VMEM((DG * WSLICE,), jnp.float32),
            pltpu.VMEM((DG * WSLICE,), jnp.float32),
            pltpu.SemaphoreType.DMA,
            pltpu.SemaphoreType.DMA,
            pltpu.SemaphoreType.DMA,
            pltpu.SemaphoreType.DMA,
        ],
        compiler_params=pltpu.CompilerParams(use_tc_tiling_on_sc=False,
                                             needs_layout_passes=False),
    )(xt, off, t2d)


def kernel(x, table):
    off = jnp.asarray(_OFF_ARG)
    trem = table[VREM0:].reshape(-1)
    tlin = _relayout(table.T, trem)
    t2d = tlin.reshape(V, D)
    out3 = _lookup(x.T, off, t2d)
    out5 = out3.reshape(F, DG, B // 128, 8, 128)
    return jnp.transpose(out5, (2, 4, 0, 1, 3)).reshape(B, F, D)


# speedup vs baseline: 2.4995x; 1.1120x over previous
import functools

import jax
import jax.numpy as jnp
import numpy as np
from jax import lax
from jax.experimental import pallas as pl
from jax.experimental.pallas import tpu as pltpu
from jax.experimental.pallas import tpu_sc as plsc

B, F, D = 16384, 26, 16
V = 2600000
_info = plsc.get_sparse_core_info()
NC, NS, L = _info.num_cores, _info.num_subcores, _info.num_lanes
NW = NC * NS

SLAB = 1024
NFULL = (V // 128) // 8
VREM0 = NFULL * SLAB
VREM = V - VREM0
APW = -(-NFULL // NW)

NFG = 2
FPG = F // NFG
NBS = NW // NFG
BPW = B // NBS
DG = D // 8
PLANE = (B // 128) * 8 * 128
WSLICE = (BPW // 128) * 8 * 128

_FIELD_DIMS = [100000] * F
_OFFSETS = np.concatenate([[0], np.cumsum(_FIELD_DIMS)[:-1]]).astype(np.int32)
_OFF_TAB = np.repeat(_OFFSETS[:, None], L, axis=1)
_BASEVEC = ((np.arange(L) // 8) * (8 * BPW)
            + (np.arange(L) % 8) * 128).astype(np.int32)
_OFF_ARG = np.concatenate([_OFF_TAB, _BASEVEC[None, :]], axis=0)


def _transpose_kernel(tt_hbm, trem_hbm, tlin_hbm, in0, in1, ob0, ob1,
                      rsem0, rsem1, wsem0, wsem1):
    wid = lax.axis_index("s") * NC + lax.axis_index("c")
    c0 = wid * APW
    end = jnp.minimum(c0 + APW, NFULL)
    iota = lax.iota(jnp.int32, L)

    ins = (in0, in1)
    obs = (ob0, ob1)
    rsems = (rsem0, rsem1)
    wsems = (wsem0, wsem1)

    def read(c, p):
        pltpu.async_copy(tt_hbm.at[:, pl.ds(c * SLAB, SLAB)], ins[p], rsems[p])

    def wait_read(p):
        pltpu.make_async_copy(tt_hbm.at[:, pl.ds(0, SLAB)], ins[p],
                              rsems[p]).wait()

    def write(c, p):
        pltpu.async_copy(obs[p], tlin_hbm.at[pl.ds(c * SLAB * D, SLAB * D)],
                         wsems[p])

    def wait_write(p):
        pltpu.make_async_copy(obs[p], tlin_hbm.at[pl.ds(0, SLAB * D)],
                              wsems[p]).wait()

    def transpose(p):
        inb, ob = ins[p], obs[p]
        @plsc.parallel_loop(0, SLAB // L, unroll=4)
        def _(j):
            jbase = iota * D + j * (L * D)
            for d in range(D):
                row = inb[d, pl.ds(j * L, L)]
                plsc.store_scatter(ob, [jbase + d], row)

    @pl.when(c0 < end)
    def _():
        read(c0, 0)

    def pair(k, carry):
        c = c0 + 2 * k

        @pl.when(c < end)
        def _():
            @pl.when(c + 1 < end)
            def _():
                read(c + 1, 1)
            wait_read(0)
            @pl.when(k > 0)
            def _():
                wait_write(0)
            transpose(0)
            write(c, 0)

        @pl.when(c + 1 < end)
        def _():
            @pl.when(c + 2 < end)
            def _():
                read(c + 2, 0)
            wait_read(1)
            @pl.when(k > 0)
            def _():
                wait_write(1)
            transpose(1)
            write(c + 1, 1)
        return carry

    lax.fori_loop(0, (APW + 1) // 2, pair, 0)

    nmine = jnp.maximum(end - c0, 0)
    @pl.when(nmine >= 1)
    def _():
        wait_write(0)
    @pl.when(nmine >= 2)
    def _():
        wait_write(1)

    @pl.when(wid == NW - 1)
    def _():
        pltpu.sync_copy(trem_hbm, ob0.at[pl.ds(0, VREM * D)])
        pltpu.sync_copy(ob0.at[pl.ds(0, VREM * D)],
                        tlin_hbm.at[pl.ds(VREM0 * D, VREM * D)])


@jax.jit
def _relayout(tt, trem):
    return pl.kernel(
        _transpose_kernel,
        out_type=jax.ShapeDtypeStruct((V * D,), jnp.float32),
        mesh=plsc.VectorSubcoreMesh(core_axis_name="c", subcore_axis_name="s"),
        scratch_types=[
            pltpu.VMEM((L, SLAB), jnp.float32),
            pltpu.VMEM((L, SLAB), jnp.float32),
            pltpu.VMEM((SLAB * D,), jnp.float32),
            pltpu.VMEM((SLAB * D,), jnp.float32),
            pltpu.SemaphoreType.DMA,
            pltpu.SemaphoreType.DMA,
            pltpu.SemaphoreType.DMA,
            pltpu.SemaphoreType.DMA,
        ],
        compiler_params=pltpu.CompilerParams(needs_layout_passes=False),
    )(tt, trem)


def _lookup_kernel(xt_hbm, off_hbm, t2d_hbm, out_hbm, idx_v, off_v, bv_v,
                   r0, r1, t0, t1, gsem0, gsem1, wsem0, wsem1):
    wid = lax.axis_index("s") * NC + lax.axis_index("c")
    fg = wid // NBS
    bs = wid % NBS
    f0 = fg * FPG
    b0 = bs * BPW
    pltpu.sync_copy(xt_hbm.at[pl.ds(f0, FPG), pl.ds(b0, BPW)], idx_v)
    pltpu.sync_copy(off_hbm.at[pl.ds(f0, FPG)], off_v)
    pltpu.sync_copy(off_hbm.at[pl.ds(F, 1)], bv_v)
    basevec = bv_v[0, :]

    def add_off(fi, carry):
        off_reg = off_v[fi, :]
        def body(j, c):
            sl = pl.ds(j * L, L)
            idx_v[fi, sl] = idx_v[fi, sl] + off_reg
            return c
        return lax.fori_loop(0, BPW // L, body, carry)

    lax.fori_loop(0, FPG, add_off, 0)

    rbufs = (r0, r1)
    tbufs = (t0, t1)
    gsems = (gsem0, gsem1)
    wsems = (wsem0, wsem1)

    def fire_gather(fi, p):
        pltpu.async_copy(t2d_hbm.at[idx_v.at[fi]], rbufs[p], gsems[p])

    def wait_gather(p):
        pltpu.make_async_copy(t2d_hbm.at[idx_v.at[0]], rbufs[p],
                              gsems[p]).wait()

    def fire_writes(fi, p):
        for dg in range(DG):
            pltpu.async_copy(
                tbufs[p].at[pl.ds(dg * WSLICE, WSLICE)],
                out_hbm.at[f0 + fi, dg, pl.ds(bs * WSLICE, WSLICE)],
                wsems[p])

    def wait_writes(p):
        for dg in range(DG):
            pltpu.make_async_copy(
                tbufs[p].at[pl.ds(dg * WSLICE, WSLICE)],
                out_hbm.at[0, dg, pl.ds(0, WSLICE)], wsems[p]).wait()

    def transpose(p):
        rb, tb = rbufs[p], tbufs[p]
        def bt_body(bt, carry):
            @plsc.parallel_loop(0, 128, unroll=4)
            def _(bl):
                row = rb[bt * 128 + bl, :]
                plsc.store_scatter(tb, [basevec + (bt * 1024 + bl)], row)
            return carry
        lax.fori_loop(0, BPW // 128, bt_body, 0)

    fire_gather(0, 0)
    for fi in range(FPG):
        p = fi % 2
        if fi + 1 < FPG:
            fire_gather(fi + 1, (fi + 1) % 2)
        wait_gather(p)
        if fi >= 2:
            wait_writes(p)
        transpose(p)
        fire_writes(fi, p)
    wait_writes((FPG - 1) % 2)
    wait_writes((FPG - 2) % 2)


@jax.jit
def _lookup(xt, off, t2d):
    return pl.kernel(
        _lookup_kernel,
        out_type=jax.ShapeDtypeStruct((F, DG, PLANE), jnp.float32),
        mesh=plsc.VectorSubcoreMesh(core_axis_name="c", subcore_axis_name="s"),
        scratch_types=[
            pltpu.VMEM((FPG, BPW), jnp.int32),
            pltpu.VMEM((FPG, L), jnp.int32),
            pltpu.VMEM((1, L), jnp.int32),
            pltpu.VMEM((BPW, D), jnp.float32),
            pltpu.VMEM((BPW, D), jnp.float32),
            pltpu.VMEM((DG * WSLICE,), jnp.float32),
            pltpu.VMEM((DG * WSLICE,), jnp.float32),
            pltpu.SemaphoreType.DMA,
            pltpu.SemaphoreType.DMA,
            pltpu.SemaphoreType.DMA,
            pltpu.SemaphoreType.DMA,
        ],
        compiler_params=pltpu.CompilerParams(use_tc_tiling_on_sc=False,
                                             needs_layout_passes=False),
    )(xt, off, t2d)


def kernel(x, table):
    off = jnp.asarray(_OFF_ARG)
    trem = table[VREM0:].reshape(-1)
    tlin = _relayout(table.T, trem)
    t2d = tlin.reshape(V, D)
    out3 = _lookup(x.T, off, t2d)
    out5 = out3.reshape(F, DG, B // 128, 8, 128)
    return jnp.transpose(out5, (2, 4, 0, 1, 3)).reshape(B, F, D)


# speedup vs baseline: 2.5956x; 1.0385x over previous
import functools

import jax
import jax.numpy as jnp
import numpy as np
from jax import lax
from jax.experimental import pallas as pl
from jax.experimental.pallas import tpu as pltpu
from jax.experimental.pallas import tpu_sc as plsc

B, F, D = 16384, 26, 16
V = 2600000
_info = plsc.get_sparse_core_info()
NC, NS, L = _info.num_cores, _info.num_subcores, _info.num_lanes
NW = NC * NS

SLAB = 1024
NFULL = (V // 128) // 8
VREM0 = NFULL * SLAB
VREM = V - VREM0
APW = -(-NFULL // NW)

NFG = 2
FPG = F // NFG
NBS = NW // NFG
BPW = B // NBS
DG = D // 8
PLANE = (B // 128) * 8 * 128
WSLICE = (BPW // 128) * 8 * 128

_FIELD_DIMS = [100000] * F
_OFFSETS = np.concatenate([[0], np.cumsum(_FIELD_DIMS)[:-1]]).astype(np.int32)
_OFF_TAB = np.repeat(_OFFSETS[:, None], L, axis=1)
_BASEVEC = ((np.arange(L) // 8) * (8 * BPW)
            + (np.arange(L) % 8) * 128).astype(np.int32)
_OFF_ARG = np.concatenate([_OFF_TAB, _BASEVEC[None, :]], axis=0)


def _transpose_kernel(tt_hbm, trem_hbm, tlin_hbm, in0, in1, ob0, ob1,
                      rsem0, rsem1, wsem0, wsem1):
    wid = lax.axis_index("s") * NC + lax.axis_index("c")
    c0 = wid * APW
    end = jnp.minimum(c0 + APW, NFULL)
    iota = lax.iota(jnp.int32, L)

    ins = (in0, in1)
    obs = (ob0, ob1)
    rsems = (rsem0, rsem1)
    wsems = (wsem0, wsem1)

    def read(c, p):
        pltpu.async_copy(tt_hbm.at[:, pl.ds(c * SLAB, SLAB)], ins[p], rsems[p])

    def wait_read(p):
        pltpu.make_async_copy(tt_hbm.at[:, pl.ds(0, SLAB)], ins[p],
                              rsems[p]).wait()

    def write(c, p):
        pltpu.async_copy(obs[p], tlin_hbm.at[pl.ds(c * SLAB * D, SLAB * D)],
                         wsems[p])

    def wait_write(p):
        pltpu.make_async_copy(obs[p], tlin_hbm.at[pl.ds(0, SLAB * D)],
                              wsems[p]).wait()

    def transpose(p):
        inb, ob = ins[p], obs[p]
        @plsc.parallel_loop(0, SLAB // L, unroll=2)
        def _(j):
            jbase = iota * D + j * (L * D)
            for d in range(D):
                row = inb[d, pl.ds(j * L, L)]
                plsc.store_scatter(ob, [jbase + d], row)

    @pl.when(c0 < end)
    def _():
        read(c0, 0)

    def pair(k, carry):
        c = c0 + 2 * k

        @pl.when(c < end)
        def _():
            @pl.when(c + 1 < end)
            def _():
                read(c + 1, 1)
            wait_read(0)
            @pl.when(k > 0)
            def _():
                wait_write(0)
            transpose(0)
            write(c, 0)

        @pl.when(c + 1 < end)
        def _():
            @pl.when(c + 2 < end)
            def _():
                read(c + 2, 0)
            wait_read(1)
            @pl.when(k > 0)
            def _():
                wait_write(1)
            transpose(1)
            write(c + 1, 1)
        return carry

    lax.fori_loop(0, (APW + 1) // 2, pair, 0)

    nmine = jnp.maximum(end - c0, 0)
    @pl.when(nmine >= 1)
    def _():
        wait_write(0)
    @pl.when(nmine >= 2)
    def _():
        wait_write(1)

    @pl.when(wid == NW - 1)
    def _():
        pltpu.sync_copy(trem_hbm, ob0.at[pl.ds(0, VREM * D)])
        pltpu.sync_copy(ob0.at[pl.ds(0, VREM * D)],
                        tlin_hbm.at[pl.ds(VREM0 * D, VREM * D)])


@jax.jit
def _relayout(tt, trem):
    return pl.kernel(
        _transpose_kernel,
        out_type=jax.ShapeDtypeStruct((V * D,), jnp.float32),
        mesh=plsc.VectorSubcoreMesh(core_axis_name="c", subcore_axis_name="s"),
        scratch_types=[
            pltpu.VMEM((L, SLAB), jnp.float32),
            pltpu.VMEM((L, SLAB), jnp.float32),
            pltpu.VMEM((SLAB * D,), jnp.float32),
            pltpu.VMEM((SLAB * D,), jnp.float32),
            pltpu.SemaphoreType.DMA,
            pltpu.SemaphoreType.DMA,
            pltpu.SemaphoreType.DMA,
            pltpu.SemaphoreType.DMA,
        ],
        compiler_params=pltpu.CompilerParams(needs_layout_passes=False),
    )(tt, trem)


def _lookup_kernel(xt_hbm, off_hbm, t2d_hbm, out_hbm, idx_v, off_v, bv_v,
                   r0, r1, t0, t1, gsem0, gsem1, wsem0, wsem1):
    wid = lax.axis_index("s") * NC + lax.axis_index("c")
    fg = wid // NBS
    bs = wid % NBS
    f0 = fg * FPG
    b0 = bs * BPW
    pltpu.sync_copy(xt_hbm.at[pl.ds(f0, FPG), pl.ds(b0, BPW)], idx_v)
    pltpu.sync_copy(off_hbm.at[pl.ds(f0, FPG)], off_v)
    pltpu.sync_copy(off_hbm.at[pl.ds(F, 1)], bv_v)
    basevec = bv_v[0, :]

    def add_off(fi, carry):
        off_reg = off_v[fi, :]
        def body(j, c):
            sl = pl.ds(j * L, L)
            idx_v[fi, sl] = idx_v[fi, sl] + off_reg
            return c
        return lax.fori_loop(0, BPW // L, body, carry)

    lax.fori_loop(0, FPG, add_off, 0)

    rbufs = (r0, r1)
    tbufs = (t0, t1)
    gsems = (gsem0, gsem1)
    wsems = (wsem0, wsem1)

    def fire_gather(fi, p):
        pltpu.async_copy(t2d_hbm.at[idx_v.at[fi]], rbufs[p], gsems[p])

    def wait_gather(p):
        pltpu.make_async_copy(t2d_hbm.at[idx_v.at[0]], rbufs[p],
                              gsems[p]).wait()

    def fire_writes(fi, p):
        for dg in range(DG):
            pltpu.async_copy(
                tbufs[p].at[pl.ds(dg * WSLICE, WSLICE)],
                out_hbm.at[f0 + fi, dg, pl.ds(bs * WSLICE, WSLICE)],
                wsems[p])

    def wait_writes(p):
        for dg in range(DG):
            pltpu.make_async_copy(
                tbufs[p].at[pl.ds(dg * WSLICE, WSLICE)],
                out_hbm.at[0, dg, pl.ds(0, WSLICE)], wsems[p]).wait()

    def transpose(p):
        rb, tb = rbufs[p], tbufs[p]
        def bt_body(bt, carry):
            @plsc.parallel_loop(0, 128, unroll=4)
            def _(bl):
                row = rb[bt * 128 + bl, :]
                plsc.store_scatter(tb, [basevec + (bt * 1024 + bl)], row)
            return carry
        lax.fori_loop(0, BPW // 128, bt_body, 0)

    fire_gather(0, 0)
    for fi in range(FPG):
        p = fi % 2
        if fi + 1 < FPG:
            fire_gather(fi + 1, (fi + 1) % 2)
        wait_gather(p)
        if fi >= 2:
            wait_writes(p)
        transpose(p)
        fire_writes(fi, p)
    wait_writes((FPG - 1) % 2)
    wait_writes((FPG - 2) % 2)


@jax.jit
def _lookup(xt, off, t2d):
    return pl.kernel(
        _lookup_kernel,
        out_type=jax.ShapeDtypeStruct((F, DG, PLANE), jnp.float32),
        mesh=plsc.VectorSubcoreMesh(core_axis_name="c", subcore_axis_name="s"),
        scratch_types=[
            pltpu.VMEM((FPG, BPW), jnp.int32),
            pltpu.VMEM((FPG, L), jnp.int32),
            pltpu.VMEM((1, L), jnp.int32),
            pltpu.VMEM((BPW, D), jnp.float32),
            pltpu.VMEM((BPW, D), jnp.float32),
            pltpu.VMEM((DG * WSLICE,), jnp.float32),
            pltpu.VMEM((DG * WSLICE,), jnp.float32),
            pltpu.SemaphoreType.DMA,
            pltpu.SemaphoreType.DMA,
            pltpu.SemaphoreType.DMA,
            pltpu.SemaphoreType.DMA,
        ],
        compiler_params=pltpu.CompilerParams(use_tc_tiling_on_sc=False,
                                             needs_layout_passes=False),
    )(xt, off, t2d)


def kernel(x, table):
    off = jnp.asarray(_OFF_ARG)
    trem = table[VREM0:].reshape(-1)
    tlin = _relayout(table.T, trem)
    t2d = tlin.reshape(V, D)
    out3 = _lookup(x.T, off, t2d)
    out5 = out3.reshape(F, DG, B // 128, 8, 128)
    return jnp.transpose(out5, (2, 4, 0, 1, 3)).reshape(B, F, D)


# speedup vs baseline: 2.6278x; 1.0124x over previous
import functools

import jax
import jax.numpy as jnp
import numpy as np
from jax import lax
from jax.experimental import pallas as pl
from jax.experimental.pallas import tpu as pltpu
from jax.experimental.pallas import tpu_sc as plsc

B, F, D = 16384, 26, 16
V = 2600000
_info = plsc.get_sparse_core_info()
NC, NS, L = _info.num_cores, _info.num_subcores, _info.num_lanes
NW = NC * NS

SLAB = 1024
NFULL = (V // 128) // 8
VREM0 = NFULL * SLAB
VREM = V - VREM0
APW = -(-NFULL // NW)

NFG = 2
FPG = F // NFG
NBS = NW // NFG
BPW = B // NBS
DG = D // 8
PLANE = (B // 128) * 8 * 128
WSLICE = (BPW // 128) * 8 * 128

_FIELD_DIMS = [100000] * F
_OFFSETS = np.concatenate([[0], np.cumsum(_FIELD_DIMS)[:-1]]).astype(np.int32)
_OFF_TAB = np.repeat(_OFFSETS[:, None], L, axis=1)
_BASEVEC = ((np.arange(L) // 8) * (8 * BPW)
            + (np.arange(L) % 8) * 128).astype(np.int32)
_OFF_ARG = np.concatenate([_OFF_TAB, _BASEVEC[None, :]], axis=0)


def _transpose_kernel(tt_hbm, trem_hbm, tlin_hbm, in0, in1, ob0, ob1,
                      rsem0, rsem1, wsem0, wsem1):
    wid = lax.axis_index("s") * NC + lax.axis_index("c")
    c0 = wid * APW
    end = jnp.minimum(c0 + APW, NFULL)
    iota = lax.iota(jnp.int32, L)

    ins = (in0, in1)
    obs = (ob0, ob1)
    rsems = (rsem0, rsem1)
    wsems = (wsem0, wsem1)

    def read(c, p):
        pltpu.async_copy(tt_hbm.at[:, pl.ds(c * SLAB, SLAB)], ins[p], rsems[p])

    def wait_read(p):
        pltpu.make_async_copy(tt_hbm.at[:, pl.ds(0, SLAB)], ins[p],
                              rsems[p]).wait()

    def write(c, p):
        pltpu.async_copy(obs[p], tlin_hbm.at[pl.ds(c * SLAB * D, SLAB * D)],
                         wsems[p])

    def wait_write(p):
        pltpu.make_async_copy(obs[p], tlin_hbm.at[pl.ds(0, SLAB * D)],
                              wsems[p]).wait()

    def transpose(p):
        inb, ob = ins[p], obs[p]
        @plsc.parallel_loop(0, SLAB // L, unroll=2)
        def _(j):
            jbase = iota * D + j * (L * D)
            for d in range(D):
                row = inb[d, pl.ds(j * L, L)]
                plsc.store_scatter(ob, [jbase + d], row)

    @pl.when(c0 < end)
    def _():
        read(c0, 0)

    def pair(k, carry):
        c = c0 + 2 * k

        @pl.when(c < end)
        def _():
            @pl.when(c + 1 < end)
            def _():
                read(c + 1, 1)
            wait_read(0)
            @pl.when(k > 0)
            def _():
                wait_write(0)
            transpose(0)
            write(c, 0)

        @pl.when(c + 1 < end)
        def _():
            @pl.when(c + 2 < end)
            def _():
                read(c + 2, 0)
            wait_read(1)
            @pl.when(k > 0)
            def _():
                wait_write(1)
            transpose(1)
            write(c + 1, 1)
        return carry

    lax.fori_loop(0, (APW + 1) // 2, pair, 0)

    nmine = jnp.maximum(end - c0, 0)
    @pl.when(nmine >= 1)
    def _():
        wait_write(0)
    @pl.when(nmine >= 2)
    def _():
        wait_write(1)

    @pl.when(wid == NW - 1)
    def _():
        pltpu.sync_copy(trem_hbm, ob0.at[pl.ds(0, VREM * D)])
        pltpu.sync_copy(ob0.at[pl.ds(0, VREM * D)],
                        tlin_hbm.at[pl.ds(VREM0 * D, VREM * D)])


@jax.jit
def _relayout(tt, trem):
    return pl.kernel(
        _transpose_kernel,
        out_type=jax.ShapeDtypeStruct((V * D,), jnp.float32),
        mesh=plsc.VectorSubcoreMesh(core_axis_name="c", subcore_axis_name="s"),
        scratch_types=[
            pltpu.VMEM((L, SLAB), jnp.float32),
            pltpu.VMEM((L, SLAB), jnp.float32),
            pltpu.VMEM((SLAB * D,), jnp.float32),
            pltpu.VMEM((SLAB * D,), jnp.float32),
            pltpu.SemaphoreType.DMA,
            pltpu.SemaphoreType.DMA,
            pltpu.SemaphoreType.DMA,
            pltpu.SemaphoreType.DMA,
        ],
        compiler_params=pltpu.CompilerParams(needs_layout_passes=False),
    )(tt, trem)


def _lookup_kernel(xt_hbm, off_hbm, t2d_hbm, out_hbm, idx_v, off_v, bv_v,
                   r0, r1, t0, t1, gsem0, gsem1, wsem0, wsem1):
    wid = lax.axis_index("s") * NC + lax.axis_index("c")
    fg = wid // NBS
    bs = wid % NBS
    f0 = fg * FPG
    b0 = bs * BPW
    pltpu.sync_copy(xt_hbm.at[pl.ds(f0, FPG), pl.ds(b0, BPW)], idx_v)
    pltpu.sync_copy(off_hbm.at[pl.ds(f0, FPG)], off_v)
    pltpu.sync_copy(off_hbm.at[pl.ds(F, 1)], bv_v)
    basevec = bv_v[0, :]

    def add_off(fi, carry):
        off_reg = off_v[fi, :]
        def body(j, c):
            sl = pl.ds(j * L, L)
            idx_v[fi, sl] = idx_v[fi, sl] + off_reg
            return c
        return lax.fori_loop(0, BPW // L, body, carry)

    lax.fori_loop(0, FPG, add_off, 0)

    rbufs = (r0, r1)
    tbufs = (t0, t1)
    gsems = (gsem0, gsem1)
    wsems = (wsem0, wsem1)

    def fire_gather(fi, p):
        pltpu.async_copy(t2d_hbm.at[idx_v.at[fi]], rbufs[p], gsems[p])

    def wait_gather(p):
        pltpu.make_async_copy(t2d_hbm.at[idx_v.at[0]], rbufs[p],
                              gsems[p]).wait()

    def fire_writes(fi, p):
        for dg in range(DG):
            pltpu.async_copy(
                tbufs[p].at[pl.ds(dg * WSLICE, WSLICE)],
                out_hbm.at[f0 + fi, dg, pl.ds(bs * WSLICE, WSLICE)],
                wsems[p])

    def wait_writes(p):
        for dg in range(DG):
            pltpu.make_async_copy(
                tbufs[p].at[pl.ds(dg * WSLICE, WSLICE)],
                out_hbm.at[0, dg, pl.ds(0, WSLICE)], wsems[p]).wait()

    def transpose(p):
        rb, tb = rbufs[p], tbufs[p]
        def bt_body(bt, carry):
            @plsc.parallel_loop(0, 128, unroll=2)
            def _(bl):
                row = rb[bt * 128 + bl, :]
                plsc.store_scatter(tb, [basevec + (bt * 1024 + bl)], row)
            return carry
        lax.fori_loop(0, BPW // 128, bt_body, 0)

    fire_gather(0, 0)
    for fi in range(FPG):
        p = fi % 2
        if fi + 1 < FPG:
            fire_gather(fi + 1, (fi + 1) % 2)
        wait_gather(p)
        if fi >= 2:
            wait_writes(p)
        transpose(p)
        fire_writes(fi, p)
    wait_writes((FPG - 1) % 2)
    wait_writes((FPG - 2) % 2)


@jax.jit
def _lookup(xt, off, t2d):
    return pl.kernel(
        _lookup_kernel,
        out_type=jax.ShapeDtypeStruct((F, DG, PLANE), jnp.float32),
        mesh=plsc.VectorSubcoreMesh(core_axis_name="c", subcore_axis_name="s"),
        scratch_types=[
            pltpu.VMEM((FPG, BPW), jnp.int32),
            pltpu.VMEM((FPG, L), jnp.int32),
            pltpu.VMEM((1, L), jnp.int32),
            pltpu.VMEM((BPW, D), jnp.float32),
            pltpu.VMEM((BPW, D), jnp.float32),
            pltpu.VMEM((DG * WSLICE,), jnp.float32),
            pltpu.VMEM((DG * WSLICE,), jnp.float32),
            pltpu.SemaphoreType.DMA,
            pltpu.SemaphoreType.DMA,
            pltpu.SemaphoreType.DMA,
            pltpu.SemaphoreType.DMA,
        ],
        compiler_params=pltpu.CompilerParams(use_tc_tiling_on_sc=False,
                                             needs_layout_passes=False),
    )(xt, off, t2d)


def kernel(x, table):
    off = jnp.asarray(_OFF_ARG)
    trem = table[VREM0:].reshape(-1)
    tlin = _relayout(table.T, trem)
    t2d = tlin.reshape(V, D)
    out3 = _lookup(x.T, off, t2d)
    out5 = out3.reshape(F, DG, B // 128, 8, 128)
    return jnp.transpose(out5, (2, 4, 0, 1, 3)).reshape(B, F, D)
